# baseline (speedup 1.0000x reference)
import functools

import jax
import jax.numpy as jnp
from jax import lax
from jax.experimental import pallas as pl
from jax.experimental.pallas import tpu as pltpu
from jax.experimental.pallas import tpu_sc as plsc

N = 10000
D = 128
OUT = 128
E = 320000

NP = 10240
EPT = E // 16
ROWS_PT = NP // 16
G = 125
CPW = E // 32 // G

BM = 2000
BMF = 1280
BK = 2048
NKB = (N + BK - 1) // BK

_PREC = lax.Precision.DEFAULT


BM2 = 1000


def _mm_body(a0_ref, a1_ref, b_ref, o0_ref, o1_ref):
    k = pl.program_id(1)

    @pl.when(k == 0)
    def _():
        o0_ref[...] = jnp.zeros_like(o0_ref)
        o1_ref[...] = jnp.zeros_like(o1_ref)

    valid = jnp.minimum(BK, N - k * BK)
    rows = lax.broadcasted_iota(jnp.int32, (BK, OUT), 0)
    b = jnp.where(rows < valid, b_ref[...], 0.0)
    o0_ref[...] += jnp.dot(a0_ref[...], b, precision=_PREC,
                           preferred_element_type=jnp.float32)
    o1_ref[...] += jnp.dot(a1_ref[...], b, precision=_PREC,
                           preferred_element_type=jnp.float32)


def _mm(a, b):
    o0, o1 = pl.pallas_call(
        _mm_body,
        grid=(N // BM, NKB),
        in_specs=[
            pl.BlockSpec((BM2, BK), lambda i, k: (2 * i, k)),
            pl.BlockSpec((BM2, BK), lambda i, k: (2 * i + 1, k)),
            pl.BlockSpec((BK, OUT), lambda i, k: (k, k * 0)),
        ],
        out_specs=[
            pl.BlockSpec((BM2, OUT), lambda i, k: (i, k * 0)),
            pl.BlockSpec((BM2, OUT), lambda i, k: (i, k * 0)),
        ],
        out_shape=[
            jax.ShapeDtypeStruct((N // 2, OUT), jnp.float32),
            jax.ShapeDtypeStruct((N // 2, OUT), jnp.float32),
        ],
        compiler_params=pltpu.CompilerParams(
            dimension_semantics=("parallel", "arbitrary")),
    )(a, a, b)
    return o0, o1


def _h_body(x_ref, wh_ref, wc_ref, h_ref, xw_ref):
    x = x_ref[...]
    h_ref[...] = jnp.maximum(
        lax.dot_general(x, wh_ref[...], (((1,), (1,)), ((), ())),
                        precision=_PREC,
                        preferred_element_type=jnp.float32), 0.0)
    xw_ref[...] = lax.dot_general(
        x, wc_ref[...], (((1,), (1,)), ((), ())), precision=_PREC,
        preferred_element_type=jnp.float32)


def _h_front(x_pad, Wh, Wc):
    return pl.pallas_call(
        _h_body,
        grid=(NP // BMF,),
        in_specs=[
            pl.BlockSpec((BMF, D), lambda i: (i, i * 0)),
            pl.BlockSpec((OUT, D), lambda i: (i * 0, i * 0)),
            pl.BlockSpec((OUT, D), lambda i: (i * 0, i * 0)),
        ],
        out_specs=[
            pl.BlockSpec((BMF, OUT), lambda i: (i, i * 0)),
            pl.BlockSpec((BMF, OUT), lambda i: (i, i * 0)),
        ],
        out_shape=[
            jax.ShapeDtypeStruct((NP, OUT), jnp.float32),
            jax.ShapeDtypeStruct((NP, OUT), jnp.float32),
        ],
    )(x_pad, Wh, Wc)


def _y_body(xw_ref, dis_ref, y_ref):
    y_ref[...] = dis_ref[...] * xw_ref[...]


def _y_scale(xw_pad, dis2d):
    return pl.pallas_call(
        _y_body,
        grid=(NP // BMF,),
        in_specs=[
            pl.BlockSpec((BMF, OUT), lambda i: (i, i * 0)),
            pl.BlockSpec((BMF, 1), lambda i: (i, i * 0)),
        ],
        out_specs=pl.BlockSpec((BMF, OUT), lambda i: (i, i * 0)),
        out_shape=jax.ShapeDtypeStruct((NP, OUT), jnp.float32),
    )(xw_pad, dis2d)


def _mm3_body(a_ref, b_ref, agg0_ref, agg1_ref, dis_ref, bc_ref, sc_ref,
              o_ref):
    k = pl.program_id(1)

    @pl.when(k == 0)
    def _():
        o_ref[...] = jnp.zeros_like(o_ref)

    valid = jnp.minimum(BK, N - k * BK)
    rows = lax.broadcasted_iota(jnp.int32, (BK, OUT), 0)
    b = jnp.where(rows < valid, b_ref[...], 0.0)
    o_ref[...] += jnp.dot(a_ref[...], b, precision=_PREC,
                          preferred_element_type=jnp.float32)

    @pl.when(k == NKB - 1)
    def _():
        aLv = sc_ref[0, 0]
        aHv = sc_ref[0, 1]
        hl = dis_ref[...] * (agg0_ref[...] + agg1_ref[...]) + bc_ref[...]
        o_ref[...] = aLv * hl + aHv * o_ref[...]


def _mm3(a, b, agg0, agg1, dis2d, bc2d, scal):
    return pl.pallas_call(
        _mm3_body,
        grid=(N // BM, NKB),
        in_specs=[
            pl.BlockSpec((BM, BK), lambda i, k: (i, k)),
            pl.BlockSpec((BK, OUT), lambda i, k: (k, k * 0)),
            pl.BlockSpec((BM, OUT), lambda i, k: (i, k * 0)),
            pl.BlockSpec((BM, OUT), lambda i, k: (i, k * 0)),
            pl.BlockSpec((BM, 1), lambda i, k: (i, k * 0)),
            pl.BlockSpec((1, OUT), lambda i, k: (k * 0, k * 0)),
            pl.BlockSpec((1, 2), lambda i, k: (k * 0, k * 0)),
        ],
        out_specs=pl.BlockSpec((BM, OUT), lambda i, k: (i, k * 0)),
        out_shape=jax.ShapeDtypeStruct((N, OUT), jnp.float32),
        compiler_params=pltpu.CompilerParams(
            dimension_semantics=("parallel", "arbitrary")),
    )(a, b, agg0, agg1, dis2d, bc2d, scal)


_MESH = plsc.VectorSubcoreMesh(core_axis_name="c", subcore_axis_name="s")


def _i32(v):
    return jnp.int32(v)


def _fori(n, body):
    lax.fori_loop(_i32(0), _i32(n), body, _i32(0))


def _rsqrt_newton(d):
    i = plsc.bitcast(d, jnp.int32)
    i = jnp.int32(0x5F3759DF) - lax.shift_right_logical(i, jnp.int32(1))
    y = plsc.bitcast(i, jnp.float32)
    for _ in range(3):
        y = y * (1.5 - 0.5 * d * y * y)
    return y


def _deg_body(dst_hbm, dis_hbm, stage_v, hist_v, red_v, dis_v, hist_sp):
    cid = lax.axis_index("c")
    sid = lax.axis_index("s")
    iota = lax.iota(jnp.int32, 16)

    @pl.when(cid == 0)
    def _core0():
        def _zero(i, carry):
            hist_v[pl.ds(i * _i32(16), 16)] = jnp.zeros((16,), jnp.float32)
            return carry
        _fori(NP // 16, _zero)

        pltpu.sync_copy(dst_hbm.at[pl.ds(sid * _i32(EPT), EPT)], stage_v)

        def _edges(e, carry):
            k = stage_v[pl.ds(e * _i32(16), 16)]
            sk, _ = plsc.sort_key_val(k, k)
            prev = sk.at[jnp.maximum(iota - _i32(1), _i32(0))].get(
                mode="promise_in_bounds")
            nxt = sk.at[jnp.minimum(iota + _i32(1), _i32(15))].get(
                mode="promise_in_bounds")
            new_run = (iota == 0) | (sk != prev)
            is_last = (iota == 15) | (sk != nxt)
            first_pos = plsc.cummax(jnp.where(new_run, iota, _i32(-1)))
            cnt = (iota - first_pos + _i32(1)).astype(jnp.float32)
            plsc.addupdate_scatter(hist_v, [sk], cnt, mask=is_last)
            return carry
        _fori(EPT // 16, _edges)

        pltpu.sync_copy(hist_v, hist_sp.at[sid])
        plsc.subcore_barrier()
        pltpu.sync_copy(
            hist_sp.at[:, pl.ds(sid * _i32(NP // 16), NP // 16)], red_v)
        for j in range(NP // 16 // 16):
            tot = red_v[0, pl.ds(j * 16, 16)]
            for t in range(1, 16):
                tot = tot + red_v[t, pl.ds(j * 16, 16)]
            deg = tot + 1.0
            dis_v[pl.ds(j * 16, 16)] = _rsqrt_newton(deg)

        pltpu.sync_copy(dis_v, dis_hbm.at[pl.ds(sid * _i32(NP // 16),
                                                NP // 16)])


@functools.partial(
    pl.kernel,
    out_type=jax.ShapeDtypeStruct((NP,), jnp.float32),
    mesh=_MESH,
    compiler_params=pltpu.CompilerParams(needs_layout_passes=False),
    scratch_types=[
        pltpu.VMEM((EPT,), jnp.int32),
        pltpu.VMEM((NP,), jnp.float32),
        pltpu.VMEM((16, NP // 16), jnp.float32),
        pltpu.VMEM((NP // 16,), jnp.float32),
        pltpu.VMEM_SHARED((16, NP), jnp.float32),
    ],
)
def _deg_kernel(dst_hbm, dis_hbm, stage_v, hist_v, red_v, dis_v,
                hist_sp):
    _deg_body(dst_hbm, dis_hbm, stage_v, hist_v, red_v, dis_v, hist_sp)


def _agg_body(s2_hbm, d2_hbm, y_hbm, zeros_hbm, agg0_hbm, agg1_hbm,
              idxs_v, idxd_v, rowb0_v, rowb1_v, gs0, gs1, ss0, ss1,
              acc_sp):
    cid = lax.axis_index("c")
    sid = lax.axis_index("s")
    g = cid * _i32(16) + sid
    HALF = CPW // 2

    @pl.when(cid == 0)
    def _():
        pltpu.sync_copy(y_hbm.at[pl.ds(sid * _i32(ROWS_PT), ROWS_PT)],
                        acc_sp.at[pl.ds(sid * _i32(ROWS_PT), ROWS_PT)])

    @pl.when(cid != 0)
    def _():
        for j in range(ROWS_PT // 128):
            pltpu.sync_copy(
                zeros_hbm,
                acc_sp.at[pl.ds(sid * _i32(ROWS_PT) + _i32(j * 128),
                                128)])

    plsc.subcore_barrier()

    def _wait_g0():
        pltpu.make_async_copy(y_hbm.at[idxs_v.at[_i32(0)]], rowb0_v, gs0).wait()

    def _wait_g1():
        pltpu.make_async_copy(y_hbm.at[idxs_v.at[_i32(0)]], rowb1_v, gs1).wait()

    def _wait_s0():
        pltpu.make_async_copy(rowb0_v, acc_sp.at[idxd_v.at[_i32(0)]],
                              ss0).wait()

    def _wait_s1():
        pltpu.make_async_copy(rowb1_v, acc_sp.at[idxd_v.at[_i32(0)]],
                              ss1).wait()

    for h in range(2):
        base = g * _i32(CPW) + _i32(h * HALF)
        pltpu.sync_copy(s2_hbm.at[pl.ds(base, HALF)], idxs_v)
        pltpu.sync_copy(d2_hbm.at[pl.ds(base, HALF)], idxd_v)
        pltpu.async_copy(y_hbm.at[idxs_v.at[_i32(0)]], rowb0_v, gs0)

        def _step(t, carry):
            j0 = t * _i32(2)
            j1 = j0 + _i32(1)

            @pl.when(t > 0)
            def _():
                _wait_s1()
            pltpu.async_copy(y_hbm.at[idxs_v.at[j1]], rowb1_v, gs1)
            _wait_g0()
            pltpu.async_copy(rowb0_v, acc_sp.at[idxd_v.at[j0]], ss0,
                             add=True)
            _wait_g1()
            pltpu.async_copy(rowb1_v, acc_sp.at[idxd_v.at[j1]], ss1,
                             add=True)

            @pl.when(t < HALF // 2 - 1)
            def _():
                _wait_s0()
                pltpu.async_copy(y_hbm.at[idxs_v.at[j0 + _i32(2)]],
                                 rowb0_v, gs0)
            return carry
        _fori(HALF // 2, _step)
        _wait_s0()
        _wait_s1()

    plsc.subcore_barrier()

    @pl.when(cid == 0)
    def _():
        pltpu.sync_copy(acc_sp.at[pl.ds(sid * _i32(ROWS_PT), ROWS_PT)],
                        agg0_hbm.at[pl.ds(sid * _i32(ROWS_PT), ROWS_PT)])

    @pl.when(cid != 0)
    def _():
        pltpu.sync_copy(acc_sp.at[pl.ds(sid * _i32(ROWS_PT), ROWS_PT)],
                        agg1_hbm.at[pl.ds(sid * _i32(ROWS_PT), ROWS_PT)])


@functools.partial(
    pl.kernel,
    out_type=(jax.ShapeDtypeStruct((NP, OUT), jnp.float32),
              jax.ShapeDtypeStruct((NP, OUT), jnp.float32)),
    mesh=_MESH,
    scratch_types=[
        pltpu.VMEM((CPW // 2, G), jnp.int32),
        pltpu.VMEM((CPW // 2, G), jnp.int32),
        pltpu.VMEM((G, OUT), jnp.float32),
        pltpu.VMEM((G, OUT), jnp.float32),
        pltpu.SemaphoreType.DMA,
        pltpu.SemaphoreType.DMA,
        pltpu.SemaphoreType.DMA,
        pltpu.SemaphoreType.DMA,
        pltpu.VMEM_SHARED((NP, OUT), jnp.float32),
    ],
)
def _agg_kernel(s2_hbm, d2_hbm, y_hbm, zeros_hbm, agg0_hbm, agg1_hbm,
                idxs_v, idxd_v, rowb0_v, rowb1_v, gs0, gs1, ss0, ss1,
                acc_sp):
    _agg_body(s2_hbm, d2_hbm, y_hbm, zeros_hbm, agg0_hbm, agg1_hbm,
              idxs_v, idxd_v, rowb0_v, rowb1_v, gs0, gs1, ss0, ss1,
              acc_sp)


def kernel(x, edge_index, lap, d_inv, Wh, Wc, bc, aL, aH):
    out_dtype = jnp.result_type(x.dtype, lap.dtype, d_inv.dtype, Wh.dtype,
                                Wc.dtype, bc.dtype, aL.dtype, aH.dtype)
    x = x.astype(jnp.float32)
    lap = lap.astype(jnp.float32)
    d_inv = d_inv.astype(jnp.float32)
    Wh = Wh.astype(jnp.float32)
    Wc = Wc.astype(jnp.float32)
    bc = bc.astype(jnp.float32)
    aL = aL.astype(jnp.float32)
    aH = aH.astype(jnp.float32)
    ei = edge_index.astype(jnp.int32)
    src = ei[0]
    dst = ei[1]

    s2 = src.reshape(E // G, G)
    d2 = dst.reshape(E // G, G)
    x_pad = jnp.pad(x, ((0, NP - N), (0, 0)))

    dis_pad = _deg_kernel(dst)
    dis2d = dis_pad.reshape(NP, 1)

    H, xw = _h_front(x_pad, Wh, Wc)
    y = _y_scale(xw, dis2d)

    zeros128 = jnp.zeros((128, OUT), jnp.float32)
    agg0, agg1 = _agg_kernel(s2, d2, y, zeros128)

    def _stitch(p):
        h0, h1 = p
        s = jnp.stack([h0.reshape(N // BM, BM2, OUT),
                       h1.reshape(N // BM, BM2, OUT)], axis=1)
        return s.reshape(N, OUT)

    t1 = _stitch(_mm(d_inv, H))
    t2 = _stitch(_mm(lap, t1))

    bc2d = bc.reshape(1, OUT)
    scal = jnp.concatenate([aL, aH]).reshape(1, 2).astype(jnp.float32)
    out = _mm3(d_inv, t2, agg0, agg1, dis2d, bc2d, scal)
    return out.astype(out_dtype)


# speedup vs baseline: 1.0179x; 1.0179x over previous
import functools

import jax
import jax.numpy as jnp
from jax import lax
from jax.experimental import pallas as pl
from jax.experimental.pallas import tpu as pltpu
from jax.experimental.pallas import tpu_sc as plsc

N = 10000
D = 128
OUT = 128
E = 320000

NP = 10240
EPT = E // 16
ROWS_PT = NP // 16
G = 125
CPW = E // 32 // G

BM = 2000
BMF = 1280
BK = 2048
NKB = (N + BK - 1) // BK

_PREC = lax.Precision.DEFAULT


def _mm_body(a_ref, b_ref, o_ref):
    k = pl.program_id(1)

    @pl.when(k == 0)
    def _():
        o_ref[...] = jnp.zeros_like(o_ref)

    valid = jnp.minimum(BK, N - k * BK)
    rows = lax.broadcasted_iota(jnp.int32, (BK, OUT), 0)
    b = jnp.where(rows < valid, b_ref[...], 0.0)
    o_ref[...] += jnp.dot(a_ref[...], b, precision=_PREC,
                          preferred_element_type=jnp.float32)


def _mm(a, b):
    return pl.pallas_call(
        _mm_body,
        grid=(N // BM, NKB),
        in_specs=[
            pl.BlockSpec((BM, BK), lambda i, k: (i, k)),
            pl.BlockSpec((BK, OUT), lambda i, k: (k, k * 0)),
        ],
        out_specs=pl.BlockSpec((BM, OUT), lambda i, k: (i, k * 0)),
        out_shape=jax.ShapeDtypeStruct((N, OUT), jnp.float32),
        compiler_params=pltpu.CompilerParams(
            dimension_semantics=("parallel", "arbitrary")),
    )(a, b)


def _h_body(x_ref, wh_ref, wc_ref, h_ref, xw_ref):
    x = x_ref[...]
    h_ref[...] = jnp.maximum(
        lax.dot_general(x, wh_ref[...], (((1,), (1,)), ((), ())),
                        precision=_PREC,
                        preferred_element_type=jnp.float32), 0.0)
    xw_ref[...] = lax.dot_general(
        x, wc_ref[...], (((1,), (1,)), ((), ())), precision=_PREC,
        preferred_element_type=jnp.float32)


def _h_front(x_pad, Wh, Wc):
    return pl.pallas_call(
        _h_body,
        grid=(NP // BMF,),
        in_specs=[
            pl.BlockSpec((BMF, D), lambda i: (i, i * 0)),
            pl.BlockSpec((OUT, D), lambda i: (i * 0, i * 0)),
            pl.BlockSpec((OUT, D), lambda i: (i * 0, i * 0)),
        ],
        out_specs=[
            pl.BlockSpec((BMF, OUT), lambda i: (i, i * 0)),
            pl.BlockSpec((BMF, OUT), lambda i: (i, i * 0)),
        ],
        out_shape=[
            jax.ShapeDtypeStruct((NP, OUT), jnp.float32),
            jax.ShapeDtypeStruct((NP, OUT), jnp.float32),
        ],
    )(x_pad, Wh, Wc)


def _y_body(xw_ref, dis_ref, y_ref):
    y_ref[...] = dis_ref[...] * xw_ref[...]


def _y_scale(xw_pad, dis2d):
    return pl.pallas_call(
        _y_body,
        grid=(NP // BMF,),
        in_specs=[
            pl.BlockSpec((BMF, OUT), lambda i: (i, i * 0)),
            pl.BlockSpec((BMF, 1), lambda i: (i, i * 0)),
        ],
        out_specs=pl.BlockSpec((BMF, OUT), lambda i: (i, i * 0)),
        out_shape=jax.ShapeDtypeStruct((NP, OUT), jnp.float32),
    )(xw_pad, dis2d)


def _mm3_body(a_ref, b_ref, agg0_ref, agg1_ref, dis_ref, bc_ref, sc_ref,
              o_ref):
    k = pl.program_id(1)

    @pl.when(k == 0)
    def _():
        o_ref[...] = jnp.zeros_like(o_ref)

    valid = jnp.minimum(BK, N - k * BK)
    rows = lax.broadcasted_iota(jnp.int32, (BK, OUT), 0)
    b = jnp.where(rows < valid, b_ref[...], 0.0)
    o_ref[...] += jnp.dot(a_ref[...], b, precision=_PREC,
                          preferred_element_type=jnp.float32)

    @pl.when(k == NKB - 1)
    def _():
        aLv = sc_ref[0, 0]
        aHv = sc_ref[0, 1]
        hl = dis_ref[...] * (agg0_ref[...] + agg1_ref[...]) + bc_ref[...]
        o_ref[...] = aLv * hl + aHv * o_ref[...]


def _mm3(a, b, agg0, agg1, dis2d, bc2d, scal):
    return pl.pallas_call(
        _mm3_body,
        grid=(N // BM, NKB),
        in_specs=[
            pl.BlockSpec((BM, BK), lambda i, k: (i, k)),
            pl.BlockSpec((BK, OUT), lambda i, k: (k, k * 0)),
            pl.BlockSpec((BM, OUT), lambda i, k: (i, k * 0)),
            pl.BlockSpec((BM, OUT), lambda i, k: (i, k * 0)),
            pl.BlockSpec((BM, 1), lambda i, k: (i, k * 0)),
            pl.BlockSpec((1, OUT), lambda i, k: (k * 0, k * 0)),
            pl.BlockSpec((1, 2), lambda i, k: (k * 0, k * 0)),
        ],
        out_specs=pl.BlockSpec((BM, OUT), lambda i, k: (i, k * 0)),
        out_shape=jax.ShapeDtypeStruct((N, OUT), jnp.float32),
        compiler_params=pltpu.CompilerParams(
            dimension_semantics=("parallel", "arbitrary")),
    )(a, b, agg0, agg1, dis2d, bc2d, scal)


_MESH = plsc.VectorSubcoreMesh(core_axis_name="c", subcore_axis_name="s")


def _i32(v):
    return jnp.int32(v)


def _fori(n, body):
    lax.fori_loop(_i32(0), _i32(n), body, _i32(0))


def _rsqrt_newton(d):
    i = plsc.bitcast(d, jnp.int32)
    i = jnp.int32(0x5F3759DF) - lax.shift_right_logical(i, jnp.int32(1))
    y = plsc.bitcast(i, jnp.float32)
    for _ in range(3):
        y = y * (1.5 - 0.5 * d * y * y)
    return y


def _deg_body(dst_hbm, dis_hbm, stage_v, hist_v, red_v, dis_v, hist_sp):
    cid = lax.axis_index("c")
    sid = lax.axis_index("s")
    iota = lax.iota(jnp.int32, 16)

    @pl.when(cid == 0)
    def _core0():
        def _zero(i, carry):
            hist_v[pl.ds(i * _i32(16), 16)] = jnp.zeros((16,), jnp.float32)
            return carry
        _fori(NP // 16, _zero)

        pltpu.sync_copy(dst_hbm.at[pl.ds(sid * _i32(EPT), EPT)], stage_v)

        def _edges(e, carry):
            k = stage_v[pl.ds(e * _i32(16), 16)]
            sk, _ = plsc.sort_key_val(k, k)
            prev = sk.at[jnp.maximum(iota - _i32(1), _i32(0))].get(
                mode="promise_in_bounds")
            nxt = sk.at[jnp.minimum(iota + _i32(1), _i32(15))].get(
                mode="promise_in_bounds")
            new_run = (iota == 0) | (sk != prev)
            is_last = (iota == 15) | (sk != nxt)
            first_pos = plsc.cummax(jnp.where(new_run, iota, _i32(-1)))
            cnt = (iota - first_pos + _i32(1)).astype(jnp.float32)
            plsc.addupdate_scatter(hist_v, [sk], cnt, mask=is_last)
            return carry
        _fori(EPT // 16, _edges)

        pltpu.sync_copy(hist_v, hist_sp.at[sid])
        plsc.subcore_barrier()
        pltpu.sync_copy(
            hist_sp.at[:, pl.ds(sid * _i32(NP // 16), NP // 16)], red_v)
        for j in range(NP // 16 // 16):
            tot = red_v[0, pl.ds(j * 16, 16)]
            for t in range(1, 16):
                tot = tot + red_v[t, pl.ds(j * 16, 16)]
            deg = tot + 1.0
            dis_v[pl.ds(j * 16, 16)] = _rsqrt_newton(deg)

        pltpu.sync_copy(dis_v, dis_hbm.at[pl.ds(sid * _i32(NP // 16),
                                                NP // 16)])


@functools.partial(
    pl.kernel,
    out_type=jax.ShapeDtypeStruct((NP,), jnp.float32),
    mesh=_MESH,
    compiler_params=pltpu.CompilerParams(needs_layout_passes=False),
    scratch_types=[
        pltpu.VMEM((EPT,), jnp.int32),
        pltpu.VMEM((NP,), jnp.float32),
        pltpu.VMEM((16, NP // 16), jnp.float32),
        pltpu.VMEM((NP // 16,), jnp.float32),
        pltpu.VMEM_SHARED((16, NP), jnp.float32),
    ],
)
def _deg_kernel(dst_hbm, dis_hbm, stage_v, hist_v, red_v, dis_v,
                hist_sp):
    _deg_body(dst_hbm, dis_hbm, stage_v, hist_v, red_v, dis_v, hist_sp)


def _agg_body(s2_hbm, d2_hbm, y_hbm, zeros_hbm, agg0_hbm, agg1_hbm,
              idxs_v, idxd_v, rowb0_v, rowb1_v, gs0, gs1, ss0, ss1,
              acc_sp):
    cid = lax.axis_index("c")
    sid = lax.axis_index("s")
    g = cid * _i32(16) + sid
    HALF = CPW // 2

    @pl.when(cid == 0)
    def _():
        pltpu.sync_copy(y_hbm.at[pl.ds(sid * _i32(ROWS_PT), ROWS_PT)],
                        acc_sp.at[pl.ds(sid * _i32(ROWS_PT), ROWS_PT)])

    @pl.when(cid != 0)
    def _():
        for j in range(ROWS_PT // 128):
            pltpu.sync_copy(
                zeros_hbm,
                acc_sp.at[pl.ds(sid * _i32(ROWS_PT) + _i32(j * 128),
                                128)])

    plsc.subcore_barrier()

    def _wait_g0():
        pltpu.make_async_copy(y_hbm.at[idxs_v.at[_i32(0)]], rowb0_v, gs0).wait()

    def _wait_g1():
        pltpu.make_async_copy(y_hbm.at[idxs_v.at[_i32(0)]], rowb1_v, gs1).wait()

    def _wait_s0():
        pltpu.make_async_copy(rowb0_v, acc_sp.at[idxd_v.at[_i32(0)]],
                              ss0).wait()

    def _wait_s1():
        pltpu.make_async_copy(rowb1_v, acc_sp.at[idxd_v.at[_i32(0)]],
                              ss1).wait()

    for h in range(2):
        base = g * _i32(CPW) + _i32(h * HALF)
        pltpu.sync_copy(s2_hbm.at[pl.ds(base, HALF)], idxs_v)
        pltpu.sync_copy(d2_hbm.at[pl.ds(base, HALF)], idxd_v)
        pltpu.async_copy(y_hbm.at[idxs_v.at[_i32(0)]], rowb0_v, gs0)

        def _step(t, carry):
            j0 = t * _i32(2)
            j1 = j0 + _i32(1)

            @pl.when(t > 0)
            def _():
                _wait_s1()
            pltpu.async_copy(y_hbm.at[idxs_v.at[j1]], rowb1_v, gs1)
            _wait_g0()
            pltpu.async_copy(rowb0_v, acc_sp.at[idxd_v.at[j0]], ss0,
                             add=True)
            _wait_g1()
            pltpu.async_copy(rowb1_v, acc_sp.at[idxd_v.at[j1]], ss1,
                             add=True)

            @pl.when(t < HALF // 2 - 1)
            def _():
                _wait_s0()
                pltpu.async_copy(y_hbm.at[idxs_v.at[j0 + _i32(2)]],
                                 rowb0_v, gs0)
            return carry
        _fori(HALF // 2, _step)
        _wait_s0()
        _wait_s1()

    plsc.subcore_barrier()

    @pl.when(cid == 0)
    def _():
        pltpu.sync_copy(acc_sp.at[pl.ds(sid * _i32(ROWS_PT), ROWS_PT)],
                        agg0_hbm.at[pl.ds(sid * _i32(ROWS_PT), ROWS_PT)])

    @pl.when(cid != 0)
    def _():
        pltpu.sync_copy(acc_sp.at[pl.ds(sid * _i32(ROWS_PT), ROWS_PT)],
                        agg1_hbm.at[pl.ds(sid * _i32(ROWS_PT), ROWS_PT)])


@functools.partial(
    pl.kernel,
    out_type=(jax.ShapeDtypeStruct((NP, OUT), jnp.float32),
              jax.ShapeDtypeStruct((NP, OUT), jnp.float32)),
    mesh=_MESH,
    scratch_types=[
        pltpu.VMEM((CPW // 2, G), jnp.int32),
        pltpu.VMEM((CPW // 2, G), jnp.int32),
        pltpu.VMEM((G, OUT), jnp.float32),
        pltpu.VMEM((G, OUT), jnp.float32),
        pltpu.SemaphoreType.DMA,
        pltpu.SemaphoreType.DMA,
        pltpu.SemaphoreType.DMA,
        pltpu.SemaphoreType.DMA,
        pltpu.VMEM_SHARED((NP, OUT), jnp.float32),
    ],
)
def _agg_kernel(s2_hbm, d2_hbm, y_hbm, zeros_hbm, agg0_hbm, agg1_hbm,
                idxs_v, idxd_v, rowb0_v, rowb1_v, gs0, gs1, ss0, ss1,
                acc_sp):
    _agg_body(s2_hbm, d2_hbm, y_hbm, zeros_hbm, agg0_hbm, agg1_hbm,
              idxs_v, idxd_v, rowb0_v, rowb1_v, gs0, gs1, ss0, ss1,
              acc_sp)


def kernel(x, edge_index, lap, d_inv, Wh, Wc, bc, aL, aH):
    out_dtype = jnp.result_type(x.dtype, lap.dtype, d_inv.dtype, Wh.dtype,
                                Wc.dtype, bc.dtype, aL.dtype, aH.dtype)
    x = x.astype(jnp.float32)
    lap = lap.astype(jnp.float32)
    d_inv = d_inv.astype(jnp.float32)
    Wh = Wh.astype(jnp.float32)
    Wc = Wc.astype(jnp.float32)
    bc = bc.astype(jnp.float32)
    aL = aL.astype(jnp.float32)
    aH = aH.astype(jnp.float32)
    ei = edge_index.astype(jnp.int32)
    src = ei[0]
    dst = ei[1]

    s2 = src.reshape(E // G, G)
    d2 = dst.reshape(E // G, G)
    x_pad = jnp.pad(x, ((0, NP - N), (0, 0)))

    dis_pad = _deg_kernel(dst)
    dis2d = dis_pad.reshape(NP, 1)

    H, xw = _h_front(x_pad, Wh, Wc)
    y = _y_scale(xw, dis2d)

    zeros128 = jnp.zeros((128, OUT), jnp.float32)
    agg0, agg1 = _agg_kernel(s2, d2, y, zeros128)

    t1 = _mm(d_inv, H)
    t2 = _mm(lap, t1)

    bc2d = bc.reshape(1, OUT)
    scal = jnp.concatenate([aL, aH]).reshape(1, 2).astype(jnp.float32)
    out = _mm3(d_inv, t2, agg0, agg1, dis2d, bc2d, scal)
    return out.astype(out_dtype)
